# baseline (speedup 1.0000x reference)
import jax
import jax.numpy as jnp
from jax.experimental import pallas as pl
from jax.experimental.pallas import tpu as pltpu


def _xw_kernel(f_ref, w_ref, o_ref):
    o_ref[:, :] = jnp.dot(f_ref[:, :], w_ref[:, :],
                          preferred_element_type=jnp.float32)


def _gcn_kernel(adj_ref, xw_ref, b_ref, o_ref):
    acc = jnp.dot(adj_ref[:, :], xw_ref[:, :],
                  preferred_element_type=jnp.float32)
    o_ref[:, :] = jnp.maximum(acc + b_ref[:, :], 0.0)


def kernel(adj, feature, W, b):
    n, f_in = feature.shape
    f_out = W.shape[1]

    xw = pl.pallas_call(
        _xw_kernel,
        out_shape=jax.ShapeDtypeStruct((n, f_out), jnp.float32),
    )(feature, W)

    bm = 400
    out = pl.pallas_call(
        _gcn_kernel,
        grid=(n // bm,),
        in_specs=[
            pl.BlockSpec((bm, n), lambda i: (i, 0)),
            pl.BlockSpec((n, f_out), lambda i: (0, 0)),
            pl.BlockSpec((1, f_out), lambda i: (0, 0)),
        ],
        out_specs=pl.BlockSpec((bm, f_out), lambda i: (i, 0)),
        out_shape=jax.ShapeDtypeStruct((n, f_out), jnp.float32),
        compiler_params=pltpu.CompilerParams(
            dimension_semantics=("parallel",)),
    )(adj, xw, b.reshape(1, f_out))
    return out


# speedup vs baseline: 1.0432x; 1.0432x over previous
import jax
import jax.numpy as jnp
from jax.experimental import pallas as pl
from jax.experimental.pallas import tpu as pltpu


def _fused_kernel(f_ref, w_ref, adj_ref, b_ref, o_ref, xw_ref):
    i = pl.program_id(0)

    @pl.when(i == 0)
    def _():
        xw_ref[:, :] = jnp.dot(f_ref[:, :], w_ref[:, :],
                               preferred_element_type=jnp.float32)

    acc = jnp.dot(adj_ref[:, :], xw_ref[:, :],
                  preferred_element_type=jnp.float32)
    o_ref[:, :] = jnp.maximum(acc + b_ref[:, :], 0.0)


def kernel(adj, feature, W, b):
    n, f_in = feature.shape
    f_out = W.shape[1]

    bm = 400
    out = pl.pallas_call(
        _fused_kernel,
        grid=(n // bm,),
        in_specs=[
            pl.BlockSpec((n, f_in), lambda i: (0, 0)),
            pl.BlockSpec((f_in, f_out), lambda i: (0, 0)),
            pl.BlockSpec((bm, n), lambda i: (i, 0)),
            pl.BlockSpec((1, f_out), lambda i: (0, 0)),
        ],
        out_specs=pl.BlockSpec((bm, f_out), lambda i: (i, 0)),
        out_shape=jax.ShapeDtypeStruct((n, f_out), jnp.float32),
        scratch_shapes=[pltpu.VMEM((n, f_out), jnp.float32)],
        compiler_params=pltpu.CompilerParams(
            dimension_semantics=("arbitrary",)),
    )(feature, W, adj, b.reshape(1, f_out))
    return out


# speedup vs baseline: 1.0445x; 1.0012x over previous
import jax
import jax.numpy as jnp
from jax.experimental import pallas as pl
from jax.experimental.pallas import tpu as pltpu


def _fused_kernel(f_ref, w_ref, adj_ref, b_ref, o_ref, xw_ref):
    i = pl.program_id(0)

    @pl.when(i == 0)
    def _():
        xw_ref[:, :] = jnp.dot(f_ref[:, :], w_ref[:, :],
                               preferred_element_type=jnp.float32
                               ).astype(jnp.bfloat16)

    acc = jnp.dot(adj_ref[:, :].astype(jnp.bfloat16), xw_ref[:, :],
                  preferred_element_type=jnp.float32)
    o_ref[:, :] = jnp.maximum(acc + b_ref[:, :], 0.0)


def kernel(adj, feature, W, b):
    n, f_in = feature.shape
    f_out = W.shape[1]

    bm = 400
    out = pl.pallas_call(
        _fused_kernel,
        grid=(n // bm,),
        in_specs=[
            pl.BlockSpec((n, f_in), lambda i: (0, 0)),
            pl.BlockSpec((f_in, f_out), lambda i: (0, 0)),
            pl.BlockSpec((bm, n), lambda i: (i, 0)),
            pl.BlockSpec((1, f_out), lambda i: (0, 0)),
        ],
        out_specs=pl.BlockSpec((bm, f_out), lambda i: (i, 0)),
        out_shape=jax.ShapeDtypeStruct((n, f_out), jnp.float32),
        scratch_shapes=[pltpu.VMEM((n, f_out), jnp.bfloat16)],
        compiler_params=pltpu.CompilerParams(
            dimension_semantics=("arbitrary",)),
    )(feature, W, adj, b.reshape(1, f_out))
    return out
